# baseline (speedup 1.0000x reference)
import functools

import jax
import jax.numpy as jnp
from jax import lax
from jax.experimental import pallas as pl
from jax.experimental.pallas import tpu as pltpu
from jax.experimental.pallas import tpu_sc as plsc

N = 10000
E = 160000
D = 256
H = 512
C = 64
G = 16

NP = 10240
EPAD = 163840
NB = 80
BK = 128
NBH = 20
BK2 = 128
ROWS_PER_TILE = NP // 16
NCH = 4
BN = 512
NI = NP // BN

_mesh = plsc.VectorSubcoreMesh(core_axis_name="c", subcore_axis_name="s")


@functools.partial(
    pl.kernel,
    mesh=_mesh,
    out_type=jax.ShapeDtypeStruct((NP, 16), jnp.float32),
    scratch_types=[
        pltpu.VMEM((NB, BK), jnp.int32),
        pltpu.VMEM((BK, 16), jnp.float32),
        pltpu.VMEM((BK, 16), jnp.float32),
        pltpu.VMEM_SHARED((NP, 16), jnp.float32),
    ],
)
def _deg_kernel(dst_hbm, deg_hbm, dst_v, ones_v, zeros_v, acc):
    cid = lax.axis_index("c")
    sid = lax.axis_index("s")

    @pl.when(cid == 0)
    def _():
        def initrow(r, _):
            ones_v[r] = jnp.ones((16,), jnp.float32)
            zeros_v[r] = jnp.zeros((16,), jnp.float32)
            return 0
        lax.fori_loop(0, BK, initrow, 0)
        base = sid * ROWS_PER_TILE
        for t in range(ROWS_PER_TILE // BK):
            pltpu.sync_copy(zeros_v, acc.at[pl.ds(base + t * BK, BK)])
        pltpu.sync_copy(dst_hbm.at[sid], dst_v)
        plsc.subcore_barrier()

        def body(j, _):
            pltpu.sync_copy(ones_v, acc.at[dst_v.at[j]], add=True)
            return 0
        lax.fori_loop(0, NB, body, 0)
        plsc.subcore_barrier()
        for t in range(ROWS_PER_TILE // BK):
            s = base + t * BK
            pltpu.sync_copy(acc.at[pl.ds(s, BK)], deg_hbm.at[pl.ds(s, BK)])


@functools.partial(
    pl.kernel,
    mesh=_mesh,
    out_type=jax.ShapeDtypeStruct((NCH * NP, 128), jnp.float32),
    scratch_types=[
        pltpu.VMEM((NBH, BK2), jnp.int32),
        pltpu.VMEM((NBH, BK2), jnp.int32),
        pltpu.VMEM((BK2, 128), jnp.float32),
        pltpu.VMEM((BK2, 128), jnp.float32),
        pltpu.VMEM_SHARED((NP, 128), jnp.float32),
        pltpu.SemaphoreType.DMA,
        pltpu.SemaphoreType.DMA,
    ],
)
def _agg_kernel(hs_hbm, src_hbm, dst_hbm, agg_hbm,
                src_v, dst_v, buf0, buf1, acc, g0, g1):
    cid = lax.axis_index("c")
    sid = lax.axis_index("s")
    base = sid * ROWS_PER_TILE

    def zrow(r, _):
        for c8 in range(8):
            sl = pl.ds(c8 * 16, 16)
            buf0[r, sl] = jnp.zeros((16,), jnp.float32)
        return 0
    lax.fori_loop(0, BK2, zrow, 0)

    for cc in range(2):
        off = cid * (2 * NP) + cc * NP
        for t in range(ROWS_PER_TILE // BK2):
            pltpu.sync_copy(buf0, acc.at[pl.ds(base + t * BK2, BK2)])
        plsc.subcore_barrier()

        for qtr in range(4):
            pltpu.sync_copy(src_hbm.at[sid * 4 + qtr], src_v)
            pltpu.sync_copy(dst_hbm.at[sid * 4 + qtr], dst_v)

            def orow(r, _):
                for c4 in range(BK2 // 16):
                    sl = pl.ds(c4 * 16, 16)
                    src_v[r, sl] = src_v[r, sl] + off
                return 0
            lax.fori_loop(0, NBH, orow, 0)

            bufs = (buf0, buf1)
            gsems = (g0, g1)
            pltpu.async_copy(hs_hbm.at[src_v.at[0]], bufs[0], gsems[0])

            def pair(p, _):
                for i in range(2):
                    j = 2 * p + i
                    jn = j + 1
                    ib = (i + 1) % 2

                    @pl.when(jn < NBH)
                    def _(jn=jn, ib=ib):
                        pltpu.async_copy(
                            hs_hbm.at[src_v.at[jn]], bufs[ib], gsems[ib])

                    pltpu.make_async_copy(
                        hs_hbm.at[src_v.at[j]], bufs[i], gsems[i]).wait()
                    pltpu.sync_copy(bufs[i], acc.at[dst_v.at[j]], add=True)
                return 0
            lax.fori_loop(0, NBH // 2, pair, 0)
        plsc.subcore_barrier()

        for t in range(ROWS_PER_TILE // BK):
            s = base + t * BK
            pltpu.sync_copy(acc.at[pl.ds(s, BK)], agg_hbm.at[pl.ds(off + s, BK)])
        plsc.subcore_barrier()

        if cc == 0:
            lax.fori_loop(0, BK2, zrow, 0)


def _mm1_body(x_ref, w_ref, deg_ref, out_ref, acc):
    k = pl.program_id(1)

    @pl.when(k == 0)
    def _():
        acc[...] = jnp.zeros_like(acc)

    acc[...] += jax.lax.dot_general(
        x_ref[...], w_ref[0], (((1,), (0,)), ((), ())),
        preferred_element_type=jnp.float32)

    @pl.when(k == D // 128 - 1)
    def _():
        dis = jax.lax.rsqrt(deg_ref[...][:, :1] + 1.0)
        hs = dis * acc[...]
        for c in range(NCH):
            out_ref[c] = hs[:, c * 128:(c + 1) * 128]


def _mm1(x_pad, w1r, deg):
    return pl.pallas_call(
        _mm1_body,
        grid=(NI, D // 128),
        in_specs=[
            pl.BlockSpec((BN, 128), lambda i, k: (i, k)),
            pl.BlockSpec((1, 128, H), lambda i, k: (k, 0, 0)),
            pl.BlockSpec((BN, 16), lambda i, k: (i, 0)),
        ],
        out_specs=pl.BlockSpec((NCH, BN, 128), lambda i, k: (0, i, 0)),
        out_shape=jax.ShapeDtypeStruct((NCH, NP, 128), jnp.float32),
        scratch_shapes=[pltpu.VMEM((BN, H), jnp.float32)],
    )(x_pad, w1r, deg)


def _mm23_body(agg_ref, hs_ref, b_ref, w_ref, deg_ref, out_ref, acc):
    k = pl.program_id(1)
    dis = jax.lax.rsqrt(deg_ref[...][:, :1] + 1.0)
    x = jnp.maximum(dis * (agg_ref[0] + hs_ref[0]) + b_ref[0, 0], 0.0)

    @pl.when(k == 0)
    def _():
        acc[...] = jnp.zeros_like(acc)

    acc[...] += jax.lax.dot_general(
        x, w_ref[0], (((1,), (0,)), ((), ())),
        preferred_element_type=jnp.float32)

    @pl.when(k == NCH - 1)
    def _():
        hs = dis * acc[...]
        for c in range(NCH):
            out_ref[c] = hs[:, c * 128:(c + 1) * 128]


def _mm23(agg, hs, br, wr, deg):
    return pl.pallas_call(
        _mm23_body,
        grid=(NI, NCH),
        in_specs=[
            pl.BlockSpec((1, BN, 128), lambda i, k: (k, i, 0)),
            pl.BlockSpec((1, BN, 128), lambda i, k: (k, i, 0)),
            pl.BlockSpec((1, 1, 128), lambda i, k: (k, 0, 0)),
            pl.BlockSpec((1, 128, H), lambda i, k: (k, 0, 0)),
            pl.BlockSpec((BN, 16), lambda i, k: (i, 0)),
        ],
        out_specs=pl.BlockSpec((NCH, BN, 128), lambda i, k: (0, i, 0)),
        out_shape=jax.ShapeDtypeStruct((NCH, NP, 128), jnp.float32),
        scratch_shapes=[pltpu.VMEM((BN, H), jnp.float32)],
    )(agg, hs, br, wr, deg)


def _final_body(agg_ref, hs_ref, b_ref, batch_ref, wl_ref, bl_ref, deg_ref,
                logits_ref, probs_ref, pooled, cnt):
    i = pl.program_id(0)
    c = pl.program_id(1)
    dis = jax.lax.rsqrt(deg_ref[...][:, :1] + 1.0)
    x3 = dis * (agg_ref[0] + hs_ref[0]) + b_ref[0, 0]
    bvec = batch_ref[0, 0]
    onehot = (jax.lax.broadcasted_iota(jnp.int32, (G, BN), 0)
              == bvec[None, :]).astype(jnp.float32)

    @pl.when(c == 0)
    def _():
        @pl.when(i == 0)
        def _():
            cnt[...] = jnp.zeros_like(cnt)
        cnt[...] += jnp.broadcast_to(
            jnp.sum(onehot, axis=1, keepdims=True), cnt.shape)

    part = jax.lax.dot_general(
        onehot, x3, (((1,), (0,)), ((), ())),
        preferred_element_type=jnp.float32)

    @pl.when(i == 0)
    def _():
        pooled[c] = part

    @pl.when(i > 0)
    def _():
        pooled[c] += part

    @pl.when(jnp.logical_and(i == NI - 1, c == NCH - 1))
    def _():
        cl = jnp.maximum(cnt[...][:, :1], 1.0)
        pm = jnp.concatenate([pooled[cc] for cc in range(NCH)], axis=1)
        mean = pm / cl
        logits = jax.lax.dot_general(
            mean, wl_ref[...], (((1,), (0,)), ((), ())),
            preferred_element_type=jnp.float32) + bl_ref[0, 0]
        logits_ref[...] = logits
        probs_ref[...] = jax.nn.softmax(logits, axis=-1)


def _final(agg, hs, b3r, batch3d, wl, bl3, deg):
    return pl.pallas_call(
        _final_body,
        grid=(NI, NCH),
        in_specs=[
            pl.BlockSpec((1, BN, 128), lambda i, c: (c, i, 0)),
            pl.BlockSpec((1, BN, 128), lambda i, c: (c, i, 0)),
            pl.BlockSpec((1, 1, 128), lambda i, c: (c, 0, 0)),
            pl.BlockSpec((1, 1, BN), lambda i, c: (i, 0, 0)),
            pl.BlockSpec((H, C), lambda i, c: (0, 0)),
            pl.BlockSpec((1, 1, C), lambda i, c: (0, 0, 0)),
            pl.BlockSpec((BN, 16), lambda i, c: (i, 0)),
        ],
        out_specs=[
            pl.BlockSpec((G, C), lambda i, c: (0, 0)),
            pl.BlockSpec((G, C), lambda i, c: (0, 0)),
        ],
        out_shape=[
            jax.ShapeDtypeStruct((G, C), jnp.float32),
            jax.ShapeDtypeStruct((G, C), jnp.float32),
        ],
        scratch_shapes=[
            pltpu.VMEM((NCH, G, 128), jnp.float32),
            pltpu.VMEM((G, 128), jnp.float32),
        ],
    )(agg, hs, b3r, batch3d, wl, bl3, deg)


def kernel(x, edge_index, batch, W1, b1, W2, b2, W3, b3, Wl, bl):
    src = edge_index[0]
    dst = edge_index[1]
    epad = EPAD - E
    src_flat = jnp.concatenate([src, jnp.full((epad,), NP - 2, jnp.int32)])
    dst_flat = jnp.concatenate([dst, jnp.full((epad,), NP - 1, jnp.int32)])
    src_r = src_flat.reshape(64, NBH, BK2)
    dst_r = dst_flat.reshape(64, NBH, BK2)
    dst_deg = dst_flat.reshape(16, NB, BK)

    x_pad = jnp.pad(x, ((0, NP - N), (0, 0)))
    batch3d = jnp.concatenate(
        [batch, jnp.full((NP - N,), G, jnp.int32)]).reshape(NI, 1, BN)

    w1r = W1.reshape(D // 128, 128, H)
    w2r = W2.reshape(NCH, 128, H)
    w3r = W3.reshape(NCH, 128, H)
    b1r = b1.reshape(NCH, 1, 128)
    b2r = b2.reshape(NCH, 1, 128)
    bl3 = bl.reshape(1, 1, C)

    deg = _deg_kernel(dst_deg)
    hs1 = _mm1(x_pad, w1r, deg)
    agg1 = _agg_kernel(hs1.reshape(NCH * NP, 128), src_r, dst_r)
    agg1 = agg1.reshape(NCH, NP, 128)
    hs2 = _mm23(agg1, hs1, b1r, w2r, deg)
    agg2 = _agg_kernel(hs2.reshape(NCH * NP, 128), src_r, dst_r)
    agg2 = agg2.reshape(NCH, NP, 128)
    hs3 = _mm23(agg2, hs2, b2r, w3r, deg)
    agg3 = _agg_kernel(hs3.reshape(NCH * NP, 128), src_r, dst_r)
    agg3 = agg3.reshape(NCH, NP, 128)
    b3r = b3.reshape(NCH, 1, 128)
    logits, probs = _final(agg3, hs3, b3r, batch3d, Wl, bl3, deg)
    return (logits, probs)


# speedup vs baseline: 1.0143x; 1.0143x over previous
import functools

import jax
import jax.numpy as jnp
from jax import lax
from jax.experimental import pallas as pl
from jax.experimental.pallas import tpu as pltpu
from jax.experimental.pallas import tpu_sc as plsc

N = 10000
E = 160000
D = 256
H = 512
C = 64
G = 16

NP = 10240
EPAD = 163840
NB = 80
BK = 128
NBH = 40
BK2 = 64
ROWS_PER_TILE = NP // 16
NCH = 4
BN = 512
NI = NP // BN

_mesh = plsc.VectorSubcoreMesh(core_axis_name="c", subcore_axis_name="s")


@functools.partial(
    pl.kernel,
    mesh=_mesh,
    out_type=jax.ShapeDtypeStruct((NP, 16), jnp.float32),
    scratch_types=[
        pltpu.VMEM((NB, BK), jnp.int32),
        pltpu.VMEM((BK, 16), jnp.float32),
        pltpu.VMEM((BK, 16), jnp.float32),
        pltpu.VMEM_SHARED((NP, 16), jnp.float32),
    ],
)
def _deg_kernel(dst_hbm, deg_hbm, dst_v, ones_v, zeros_v, acc):
    cid = lax.axis_index("c")
    sid = lax.axis_index("s")

    @pl.when(cid == 0)
    def _():
        def initrow(r, _):
            ones_v[r] = jnp.ones((16,), jnp.float32)
            zeros_v[r] = jnp.zeros((16,), jnp.float32)
            return 0
        lax.fori_loop(0, BK, initrow, 0)
        base = sid * ROWS_PER_TILE
        for t in range(ROWS_PER_TILE // BK):
            pltpu.sync_copy(zeros_v, acc.at[pl.ds(base + t * BK, BK)])
        pltpu.sync_copy(dst_hbm.at[sid], dst_v)
        plsc.subcore_barrier()

        def body(j, _):
            pltpu.sync_copy(ones_v, acc.at[dst_v.at[j]], add=True)
            return 0
        lax.fori_loop(0, NB, body, 0)
        plsc.subcore_barrier()
        for t in range(ROWS_PER_TILE // BK):
            s = base + t * BK
            pltpu.sync_copy(acc.at[pl.ds(s, BK)], deg_hbm.at[pl.ds(s, BK)])


@functools.partial(
    pl.kernel,
    mesh=_mesh,
    out_type=jax.ShapeDtypeStruct((NCH * NP, 128), jnp.float32),
    scratch_types=[
        pltpu.VMEM((NBH, BK2), jnp.int32),
        pltpu.VMEM((NBH, BK2), jnp.int32),
        pltpu.VMEM((BK2, 128), jnp.float32),
        pltpu.VMEM((BK2, 128), jnp.float32),
        pltpu.VMEM((BK2, 128), jnp.float32),
        pltpu.VMEM((BK2, 128), jnp.float32),
        pltpu.VMEM_SHARED((NP, 128), jnp.float32),
        pltpu.SemaphoreType.DMA,
        pltpu.SemaphoreType.DMA,
        pltpu.SemaphoreType.DMA,
        pltpu.SemaphoreType.DMA,
        pltpu.SemaphoreType.DMA,
    ],
)
def _agg_kernel(hs_hbm, src_hbm, dst_hbm, agg_hbm,
                src_v, dst_v, buf0, buf1, buf2, buf3, acc,
                g0, g1, g2, g3, ssem):
    cid = lax.axis_index("c")
    sid = lax.axis_index("s")
    base = sid * ROWS_PER_TILE

    def zrow(r, _):
        for c8 in range(8):
            sl = pl.ds(c8 * 16, 16)
            buf0[r, sl] = jnp.zeros((16,), jnp.float32)
        return 0
    lax.fori_loop(0, BK2, zrow, 0)

    for cc in range(2):
        off = cid * (2 * NP) + cc * NP
        for t in range(ROWS_PER_TILE // BK2):
            pltpu.sync_copy(buf0, acc.at[pl.ds(base + t * BK2, BK2)])
        plsc.subcore_barrier()

        for qtr in range(4):
            pltpu.sync_copy(src_hbm.at[sid * 4 + qtr], src_v)
            pltpu.sync_copy(dst_hbm.at[sid * 4 + qtr], dst_v)

            def orow(r, _):
                for c4 in range(BK2 // 16):
                    sl = pl.ds(c4 * 16, 16)
                    src_v[r, sl] = src_v[r, sl] + off
                return 0
            lax.fori_loop(0, NBH, orow, 0)

            bufs = (buf0, buf1, buf2, buf3)
            gsems = (g0, g1, g2, g3)
            for i in range(3):
                pltpu.async_copy(hs_hbm.at[src_v.at[i]], bufs[i], gsems[i])

            def quad(p, _):
                for i in range(4):
                    j = 4 * p + i
                    jn = j + 3
                    ib = (i + 3) % 4

                    @pl.when(j >= 1)
                    def _(j=j, ib=ib):
                        pltpu.make_async_copy(
                            bufs[ib], acc.at[dst_v.at[j - 1]], ssem).wait()

                    @pl.when(jn < NBH)
                    def _(jn=jn, ib=ib):
                        pltpu.async_copy(
                            hs_hbm.at[src_v.at[jn]], bufs[ib], gsems[ib])

                    pltpu.make_async_copy(
                        hs_hbm.at[src_v.at[j]], bufs[i], gsems[i]).wait()
                    pltpu.async_copy(
                        bufs[i], acc.at[dst_v.at[j]], ssem, add=True)
                return 0
            lax.fori_loop(0, NBH // 4, quad, 0)
            pltpu.make_async_copy(
                bufs[(NBH - 1) % 4], acc.at[dst_v.at[NBH - 1]], ssem).wait()
        plsc.subcore_barrier()

        for t in range(ROWS_PER_TILE // BK):
            s = base + t * BK
            pltpu.sync_copy(acc.at[pl.ds(s, BK)], agg_hbm.at[pl.ds(off + s, BK)])
        plsc.subcore_barrier()

        if cc == 0:
            lax.fori_loop(0, BK2, zrow, 0)


def _mm1_body(x_ref, w_ref, deg_ref, out_ref, acc):
    k = pl.program_id(1)

    @pl.when(k == 0)
    def _():
        acc[...] = jnp.zeros_like(acc)

    acc[...] += jax.lax.dot_general(
        x_ref[...], w_ref[0], (((1,), (0,)), ((), ())),
        preferred_element_type=jnp.float32)

    @pl.when(k == D // 128 - 1)
    def _():
        dis = jax.lax.rsqrt(deg_ref[...][:, :1] + 1.0)
        hs = dis * acc[...]
        for c in range(NCH):
            out_ref[c] = hs[:, c * 128:(c + 1) * 128]


def _mm1(x_pad, w1r, deg):
    return pl.pallas_call(
        _mm1_body,
        grid=(NI, D // 128),
        in_specs=[
            pl.BlockSpec((BN, 128), lambda i, k: (i, k)),
            pl.BlockSpec((1, 128, H), lambda i, k: (k, 0, 0)),
            pl.BlockSpec((BN, 16), lambda i, k: (i, 0)),
        ],
        out_specs=pl.BlockSpec((NCH, BN, 128), lambda i, k: (0, i, 0)),
        out_shape=jax.ShapeDtypeStruct((NCH, NP, 128), jnp.float32),
        scratch_shapes=[pltpu.VMEM((BN, H), jnp.float32)],
    )(x_pad, w1r, deg)


def _mm23_body(agg_ref, hs_ref, b_ref, w_ref, deg_ref, out_ref, acc):
    k = pl.program_id(1)
    dis = jax.lax.rsqrt(deg_ref[...][:, :1] + 1.0)
    x = jnp.maximum(dis * (agg_ref[0] + hs_ref[0]) + b_ref[0, 0], 0.0)

    @pl.when(k == 0)
    def _():
        acc[...] = jnp.zeros_like(acc)

    acc[...] += jax.lax.dot_general(
        x, w_ref[0], (((1,), (0,)), ((), ())),
        preferred_element_type=jnp.float32)

    @pl.when(k == NCH - 1)
    def _():
        hs = dis * acc[...]
        for c in range(NCH):
            out_ref[c] = hs[:, c * 128:(c + 1) * 128]


def _mm23(agg, hs, br, wr, deg):
    return pl.pallas_call(
        _mm23_body,
        grid=(NI, NCH),
        in_specs=[
            pl.BlockSpec((1, BN, 128), lambda i, k: (k, i, 0)),
            pl.BlockSpec((1, BN, 128), lambda i, k: (k, i, 0)),
            pl.BlockSpec((1, 1, 128), lambda i, k: (k, 0, 0)),
            pl.BlockSpec((1, 128, H), lambda i, k: (k, 0, 0)),
            pl.BlockSpec((BN, 16), lambda i, k: (i, 0)),
        ],
        out_specs=pl.BlockSpec((NCH, BN, 128), lambda i, k: (0, i, 0)),
        out_shape=jax.ShapeDtypeStruct((NCH, NP, 128), jnp.float32),
        scratch_shapes=[pltpu.VMEM((BN, H), jnp.float32)],
    )(agg, hs, br, wr, deg)


def _final_body(agg_ref, hs_ref, b_ref, batch_ref, wl_ref, bl_ref, deg_ref,
                logits_ref, probs_ref, pooled, cnt):
    i = pl.program_id(0)
    c = pl.program_id(1)
    dis = jax.lax.rsqrt(deg_ref[...][:, :1] + 1.0)
    x3 = dis * (agg_ref[0] + hs_ref[0]) + b_ref[0, 0]
    bvec = batch_ref[0, 0]
    onehot = (jax.lax.broadcasted_iota(jnp.int32, (G, BN), 0)
              == bvec[None, :]).astype(jnp.float32)

    @pl.when(c == 0)
    def _():
        @pl.when(i == 0)
        def _():
            cnt[...] = jnp.zeros_like(cnt)
        cnt[...] += jnp.broadcast_to(
            jnp.sum(onehot, axis=1, keepdims=True), cnt.shape)

    part = jax.lax.dot_general(
        onehot, x3, (((1,), (0,)), ((), ())),
        preferred_element_type=jnp.float32)

    @pl.when(i == 0)
    def _():
        pooled[c] = part

    @pl.when(i > 0)
    def _():
        pooled[c] += part

    @pl.when(jnp.logical_and(i == NI - 1, c == NCH - 1))
    def _():
        cl = jnp.maximum(cnt[...][:, :1], 1.0)
        pm = jnp.concatenate([pooled[cc] for cc in range(NCH)], axis=1)
        mean = pm / cl
        logits = jax.lax.dot_general(
            mean, wl_ref[...], (((1,), (0,)), ((), ())),
            preferred_element_type=jnp.float32) + bl_ref[0, 0]
        logits_ref[...] = logits
        probs_ref[...] = jax.nn.softmax(logits, axis=-1)


def _final(agg, hs, b3r, batch3d, wl, bl3, deg):
    return pl.pallas_call(
        _final_body,
        grid=(NI, NCH),
        in_specs=[
            pl.BlockSpec((1, BN, 128), lambda i, c: (c, i, 0)),
            pl.BlockSpec((1, BN, 128), lambda i, c: (c, i, 0)),
            pl.BlockSpec((1, 1, 128), lambda i, c: (c, 0, 0)),
            pl.BlockSpec((1, 1, BN), lambda i, c: (i, 0, 0)),
            pl.BlockSpec((H, C), lambda i, c: (0, 0)),
            pl.BlockSpec((1, 1, C), lambda i, c: (0, 0, 0)),
            pl.BlockSpec((BN, 16), lambda i, c: (i, 0)),
        ],
        out_specs=[
            pl.BlockSpec((G, C), lambda i, c: (0, 0)),
            pl.BlockSpec((G, C), lambda i, c: (0, 0)),
        ],
        out_shape=[
            jax.ShapeDtypeStruct((G, C), jnp.float32),
            jax.ShapeDtypeStruct((G, C), jnp.float32),
        ],
        scratch_shapes=[
            pltpu.VMEM((NCH, G, 128), jnp.float32),
            pltpu.VMEM((G, 128), jnp.float32),
        ],
    )(agg, hs, b3r, batch3d, wl, bl3, deg)


def kernel(x, edge_index, batch, W1, b1, W2, b2, W3, b3, Wl, bl):
    src = edge_index[0]
    dst = edge_index[1]
    epad = EPAD - E
    src_flat = jnp.concatenate([src, jnp.full((epad,), NP - 2, jnp.int32)])
    dst_flat = jnp.concatenate([dst, jnp.full((epad,), NP - 1, jnp.int32)])
    src_r = src_flat.reshape(64, NBH, BK2)
    dst_r = dst_flat.reshape(64, NBH, BK2)
    dst_deg = dst_flat.reshape(16, NB, BK)

    x_pad = jnp.pad(x, ((0, NP - N), (0, 0)))
    batch3d = jnp.concatenate(
        [batch, jnp.full((NP - N,), G, jnp.int32)]).reshape(NI, 1, BN)

    w1r = W1.reshape(D // 128, 128, H)
    w2r = W2.reshape(NCH, 128, H)
    w3r = W3.reshape(NCH, 128, H)
    b1r = b1.reshape(NCH, 1, 128)
    b2r = b2.reshape(NCH, 1, 128)
    bl3 = bl.reshape(1, 1, C)

    deg = _deg_kernel(dst_deg)
    hs1 = _mm1(x_pad, w1r, deg)
    agg1 = _agg_kernel(hs1.reshape(NCH * NP, 128), src_r, dst_r)
    agg1 = agg1.reshape(NCH, NP, 128)
    hs2 = _mm23(agg1, hs1, b1r, w2r, deg)
    agg2 = _agg_kernel(hs2.reshape(NCH * NP, 128), src_r, dst_r)
    agg2 = agg2.reshape(NCH, NP, 128)
    hs3 = _mm23(agg2, hs2, b2r, w3r, deg)
    agg3 = _agg_kernel(hs3.reshape(NCH * NP, 128), src_r, dst_r)
    agg3 = agg3.reshape(NCH, NP, 128)
    b3r = b3.reshape(NCH, 1, 128)
    logits, probs = _final(agg3, hs3, b3r, batch3d, Wl, bl3, deg)
    return (logits, probs)
